# baseline (speedup 1.0000x reference)
import functools

import jax
import jax.numpy as jnp
from jax.experimental import pallas as pl
from jax.experimental.pallas import tpu as pltpu

K = 8


def _sse_kernel(x_ref, W_sel_ref, b_sel_ref, W_q_ref, b_q_ref, W_k_ref,
                b_k_ref, W_v_hbm, b_v_ref, W_o_hbm, b_o_ref, out_ref,
                wv_s, wo_s, sem_v, sem_o, *, B, S, d, P, c):
    f32 = jnp.float32
    bf16 = jnp.bfloat16
    cp_v = pltpu.make_async_copy(W_v_hbm, wv_s, sem_v)
    cp_o = pltpu.make_async_copy(W_o_hbm, wo_s, sem_o)
    cp_v.start()
    cp_o.start()
    x = x_ref[...].reshape(B * S, d)

    mm = functools.partial(jnp.dot, preferred_element_type=f32)

    def mm_tt(a, b):
        return jax.lax.dot_general(a, b, (((1,), (1,)), ((), ())),
                                   preferred_element_type=f32)

    def mm_00(a, b):
        return jax.lax.dot_general(a, b, (((0,), (0,)), ((), ())),
                                   preferred_element_type=f32)

    logitsT = jax.lax.dot_general(
        W_sel_ref[...], x, (((0,), (1,)), ((), ())),
        preferred_element_type=f32) + b_sel_ref[...]
    q = mm(x, W_q_ref[...]) + b_q_ref[...]
    kk = mm(x, W_k_ref[...]) + b_k_ref[...]

    kmax = jnp.max(kk, axis=1, keepdims=True)
    ke = jnp.exp(kk - kmax)
    w = ke / jnp.sum(ke, axis=1, keepdims=True)

    iota_p = jax.lax.broadcasted_iota(jnp.int32, (P, B * S), 0).astype(f32)
    lg = logitsT
    neg_inf = jnp.float32(-jnp.inf)
    big = jnp.float32(P)
    ohs = []
    for _ in range(K):
        m = jnp.max(lg, axis=0, keepdims=True)
        first = jnp.min(jnp.where(lg >= m, iota_p, big), axis=0, keepdims=True)
        oh = (iota_p == first)
        ohs.append(oh.astype(bf16))
        lg = jnp.where(oh, neg_inf, lg)
    A = ohs[0]
    for k in range(1, K):
        A = A + ohs[k]

    causal = (jax.lax.broadcasted_iota(jnp.int32, (S, S), 0)
              >= jax.lax.broadcasted_iota(jnp.int32, (S, S), 1)).astype(f32)
    cscale = causal * (jnp.float32(1.0) / jnp.sqrt(jnp.float32(d)))

    cp_v.wait()
    v = mm(x, wv_s[...]) + b_v_ref[...]

    reads = []
    for b in range(B):
        sl = slice(b * S, (b + 1) * S)
        qb, vb, wb = q[sl], v[sl], w[sl]
        Ab = A[:, sl]
        QVc = mm_tt(qb, vb) * cscale
        ohcat = jnp.concatenate([ohs[k][:, sl] for k in range(K)], axis=1)
        masksall = mm_00(ohcat, Ab)
        masks = [masksall[k * S:(k + 1) * S] for k in range(K)]
        scores = jnp.concatenate(
            [mm(masks[k] * QVc, wb) for k in range(K)], axis=1)
        smax = jnp.max(scores, axis=1, keepdims=True)
        se = jnp.exp(scores - smax)
        inv_den = jnp.float32(1.0) / jnp.sum(se, axis=1, keepdims=True)
        coeff = masks[0] * mm_tt(se[:, 0:c], wb)
        for k in range(1, K):
            coeff = coeff + masks[k] * mm_tt(se[:, k * c:(k + 1) * c], wb)
        coeff = coeff * (causal * inv_den)
        reads.append(mm(coeff, vb))
    read = jnp.concatenate(reads, axis=0)
    cp_o.wait()
    out = mm(read, wo_s[...]) + b_o_ref[...]
    out_ref[...] = out.reshape(B, S, d)


def kernel(x, W_sel, b_sel, W_q, b_q, W_k, b_k, W_v, b_v, W_o, b_o):
    B, S, d = x.shape
    P = W_sel.shape[1]
    c = W_k.shape[1]
    grid_kernel = functools.partial(_sse_kernel, B=B, S=S, d=d, P=P, c=c)
    vmem = pl.BlockSpec(memory_space=pltpu.VMEM)
    hbm = pl.BlockSpec(memory_space=pltpu.MemorySpace.HBM)
    return pl.pallas_call(
        grid_kernel,
        in_specs=[vmem] * 7 + [hbm, vmem, hbm, vmem],
        out_specs=vmem,
        out_shape=jax.ShapeDtypeStruct((B, S, d), jnp.float32),
        scratch_shapes=[pltpu.VMEM((d, d), jnp.float32),
                        pltpu.VMEM((d, d), jnp.float32),
                        pltpu.SemaphoreType.DMA,
                        pltpu.SemaphoreType.DMA],
    )(x, W_sel, b_sel.reshape(P, 1), W_q, b_q.reshape(1, d),
      W_k, b_k.reshape(1, c), W_v, b_v.reshape(1, d),
      W_o, b_o.reshape(1, d))


# speedup vs baseline: 1.0768x; 1.0768x over previous
import functools

import jax
import jax.numpy as jnp
from jax.experimental import pallas as pl
from jax.experimental.pallas import tpu as pltpu

K = 8


def _sse_kernel(x_ref, W_sel_ref, b_sel_ref, W_q_ref, b_q_ref, W_k_ref,
                b_k_ref, W_v_ref, b_v_ref, W_o_hbm, b_o_ref, out_ref,
                wo_s, sem_o, *, B, S, d, P, c):
    f32 = jnp.float32
    bf16 = jnp.bfloat16
    cp_o = pltpu.make_async_copy(W_o_hbm, wo_s, sem_o)
    cp_o.start()
    x = x_ref[...].reshape(B * S, d)

    mm = functools.partial(jnp.dot, preferred_element_type=f32)

    def mm_tt(a, b):
        return jax.lax.dot_general(a, b, (((1,), (1,)), ((), ())),
                                   preferred_element_type=f32)

    def mm_00(a, b):
        return jax.lax.dot_general(a, b, (((0,), (0,)), ((), ())),
                                   preferred_element_type=f32)

    logitsT = jax.lax.dot_general(
        W_sel_ref[...], x, (((0,), (1,)), ((), ())),
        preferred_element_type=f32) + b_sel_ref[...]
    q = mm(x, W_q_ref[...]) + b_q_ref[...]
    kk = mm(x, W_k_ref[...]) + b_k_ref[...]
    v = mm(x, W_v_ref[...]) + b_v_ref[...]

    kmax = jnp.max(kk, axis=1, keepdims=True)
    ke = jnp.exp(kk - kmax)
    w = ke / jnp.sum(ke, axis=1, keepdims=True)

    iota_p = jax.lax.broadcasted_iota(jnp.int32, (P, B * S), 0).astype(f32)
    lg = logitsT
    neg_inf = jnp.float32(-jnp.inf)
    big = jnp.float32(P)
    ohs = []
    for _ in range(K):
        m = jnp.max(lg, axis=0, keepdims=True)
        first = jnp.min(jnp.where(lg >= m, iota_p, big), axis=0, keepdims=True)
        oh = (iota_p == first)
        ohs.append(oh.astype(bf16))
        lg = jnp.where(oh, neg_inf, lg)
    A = ohs[0]
    for k in range(1, K):
        A = A + ohs[k]

    causal = (jax.lax.broadcasted_iota(jnp.int32, (S, S), 0)
              >= jax.lax.broadcasted_iota(jnp.int32, (S, S), 1)).astype(f32)
    cscale = causal * (jnp.float32(1.0) / jnp.sqrt(jnp.float32(d)))

    reads = []
    for b in range(B):
        sl = slice(b * S, (b + 1) * S)
        qb, vb, wb = q[sl], v[sl], w[sl]
        Ab = A[:, sl]
        QVc = mm_tt(qb, vb) * cscale
        ohcat = jnp.concatenate([ohs[k][:, sl] for k in range(K)], axis=1)
        masksall = mm_00(ohcat, Ab)
        masks = [masksall[k * S:(k + 1) * S] for k in range(K)]
        scores = jnp.concatenate(
            [mm(masks[k] * QVc, wb) for k in range(K)], axis=1)
        smax = jnp.max(scores, axis=1, keepdims=True)
        se = jnp.exp(scores - smax)
        inv_den = jnp.float32(1.0) / jnp.sum(se, axis=1, keepdims=True)
        coeff = masks[0] * mm_tt(se[:, 0:c], wb)
        for k in range(1, K):
            coeff = coeff + masks[k] * mm_tt(se[:, k * c:(k + 1) * c], wb)
        coeff = coeff * (causal * inv_den)
        reads.append(mm(coeff, vb))
    read = jnp.concatenate(reads, axis=0)
    cp_o.wait()
    out = mm(read, wo_s[...]) + b_o_ref[...]
    out_ref[...] = out.reshape(B, S, d)


def kernel(x, W_sel, b_sel, W_q, b_q, W_k, b_k, W_v, b_v, W_o, b_o):
    B, S, d = x.shape
    P = W_sel.shape[1]
    c = W_k.shape[1]
    grid_kernel = functools.partial(_sse_kernel, B=B, S=S, d=d, P=P, c=c)
    vmem = pl.BlockSpec(memory_space=pltpu.VMEM)
    hbm = pl.BlockSpec(memory_space=pltpu.MemorySpace.HBM)
    return pl.pallas_call(
        grid_kernel,
        in_specs=[vmem] * 9 + [hbm, vmem],
        out_specs=vmem,
        out_shape=jax.ShapeDtypeStruct((B, S, d), jnp.float32),
        scratch_shapes=[pltpu.VMEM((d, d), jnp.float32),
                        pltpu.SemaphoreType.DMA],
    )(x, W_sel, b_sel.reshape(P, 1), W_q, b_q.reshape(1, d),
      W_k, b_k.reshape(1, c), W_v, b_v.reshape(1, d),
      W_o, b_o.reshape(1, d))
